# baseline (speedup 1.0000x reference)
import jax
import jax.numpy as jnp
from jax import lax
from jax.experimental import pallas as pl
from jax.experimental.pallas import tpu as pltpu
from jax.experimental.pallas import tpu_sc as plsc

ENTITY_NUM = 1000000
RELATION_NUM = 1000
EMBED_DIM = 64
BATCH = 16384

NUM_CORES = 2
NUM_SUBCORES = 16
NUM_WORKERS = NUM_CORES * NUM_SUBCORES
ROWS_PER_WORKER = BATCH // NUM_WORKERS
LANES = 16
VECS_PER_ROW = EMBED_DIM // LANES


def _transe_body(ent_hbm, rel_hbm, heads_hbm, rels_hbm, tails_hbm, out_hbm,
                 hidx, ridx, tidx, hrows, rrows, trows, sem_h, sem_r, sem_t):
    wid = lax.axis_index("s") * NUM_CORES + lax.axis_index("c")
    base = wid * ROWS_PER_WORKER

    pltpu.sync_copy(heads_hbm.at[pl.ds(base, ROWS_PER_WORKER)], hidx)
    pltpu.sync_copy(rels_hbm.at[pl.ds(base, ROWS_PER_WORKER)], ridx)
    pltpu.sync_copy(tails_hbm.at[pl.ds(base, ROWS_PER_WORKER)], tidx)

    ch = pltpu.async_copy(ent_hbm.at[hidx], hrows, sem_h)
    cr = pltpu.async_copy(rel_hbm.at[ridx], rrows, sem_r)
    ct = pltpu.async_copy(ent_hbm.at[tidx], trows, sem_t)
    ch.wait()
    cr.wait()
    ct.wait()

    def row(i, carry):
        for j in range(VECS_PER_ROW):
            sl = pl.ds(j * LANES, LANES)
            hrows[i, sl] = hrows[i, sl] + rrows[i, sl] - trows[i, sl]
        return carry

    lax.fori_loop(0, ROWS_PER_WORKER, row, 0)

    pltpu.sync_copy(hrows, out_hbm.at[pl.ds(base, ROWS_PER_WORKER)])


_transe = pl.kernel(
    _transe_body,
    out_type=jax.ShapeDtypeStruct((BATCH, EMBED_DIM), jnp.float32),
    mesh=plsc.VectorSubcoreMesh(
        core_axis_name="c", subcore_axis_name="s",
        num_cores=NUM_CORES, num_subcores=NUM_SUBCORES),
    scratch_types=[
        pltpu.VMEM((ROWS_PER_WORKER,), jnp.int32),
        pltpu.VMEM((ROWS_PER_WORKER,), jnp.int32),
        pltpu.VMEM((ROWS_PER_WORKER,), jnp.int32),
        pltpu.VMEM((ROWS_PER_WORKER, EMBED_DIM), jnp.float32),
        pltpu.VMEM((ROWS_PER_WORKER, EMBED_DIM), jnp.float32),
        pltpu.VMEM((ROWS_PER_WORKER, EMBED_DIM), jnp.float32),
        pltpu.SemaphoreType.DMA,
        pltpu.SemaphoreType.DMA,
        pltpu.SemaphoreType.DMA,
    ],
    compiler_params=pltpu.CompilerParams(use_tc_tiling_on_sc=False),
)


@jax.jit
def kernel(entity_emb, relation_emb, heads, relations, tails):
    return _transe(
        entity_emb,
        relation_emb,
        heads.astype(jnp.int32),
        relations.astype(jnp.int32),
        tails.astype(jnp.int32),
    )
